# Initial kernel scaffold; baseline (speedup 1.0000x reference)
import jax
import jax.numpy as jnp
from jax.experimental import pallas as pl


def kernel(x_batch, emb_table, fc_w, fc_b):
    raise NotImplementedError("write your pallas kernel here")


# speedup vs baseline: 34.9598x; 34.9598x over previous
import functools

import jax
import jax.numpy as jnp
from jax import lax
from jax.experimental import pallas as pl
from jax.experimental.pallas import tpu as pltpu
from jax.experimental.pallas import tpu_sc as plsc

VOCAB = 100000
EMBED_DIM = 64
NUM_CLASSES = 4
BATCH = 16384
SEQ = 200

LANES = 16
NUM_CORES = 2
NUM_SUBCORES = 16
NUM_WORKERS = NUM_CORES * NUM_SUBCORES
ROWS_PER_W = BATCH // NUM_WORKERS
GROUP = 16
NGROUPS = ROWS_PER_W // GROUP
IDX_PER_GROUP = GROUP * SEQ
STREAM_LEN = 128
NSTREAMS = IDX_PER_GROUP // STREAM_LEN

_PROJ_BLOCK = 10000


def _project_body(e_ref, w_ref, b_ref, o_ref):
    o_ref[...] = (
        jnp.dot(e_ref[...], w_ref[...], preferred_element_type=jnp.float32)
        + b_ref[...]
    )


def _project(emb_table, w16, b16):
    grid = (VOCAB // _PROJ_BLOCK,)
    return pl.pallas_call(
        _project_body,
        grid=grid,
        in_specs=[
            pl.BlockSpec((_PROJ_BLOCK, EMBED_DIM), lambda i: (i, 0)),
            pl.BlockSpec((EMBED_DIM, LANES), lambda i: (0, 0)),
            pl.BlockSpec((1, LANES), lambda i: (0, 0)),
        ],
        out_specs=pl.BlockSpec((_PROJ_BLOCK, LANES), lambda i: (i, 0)),
        out_shape=jax.ShapeDtypeStruct((VOCAB, LANES), jnp.float32),
    )(emb_table, w16, b16)


_sc_mesh = plsc.VectorSubcoreMesh(core_axis_name="c", subcore_axis_name="s")


@functools.partial(
    pl.kernel,
    mesh=_sc_mesh,
    compiler_params=pltpu.CompilerParams(use_tc_tiling_on_sc=False),
    out_type=jax.ShapeDtypeStruct((BATCH * NUM_CLASSES,), jnp.float32),
    scratch_types=[
        pltpu.VMEM((IDX_PER_GROUP,), jnp.int32),
        pltpu.VMEM((IDX_PER_GROUP, LANES), jnp.float32),
        pltpu.VMEM((ROWS_PER_W * NUM_CLASSES,), jnp.float32),
        pltpu.SemaphoreType.DMA,
    ],
)
def _pool_kernel(text_hbm, p_hbm, out_hbm, idx_v, rows_v, out_v, sem):
    wid = lax.axis_index("s") * NUM_CORES + lax.axis_index("c")
    tok_base = wid * (ROWS_PER_W * SEQ)

    lane = lax.iota(jnp.int32, LANES)
    m0 = lane < 4
    m1 = lane < 8
    m2 = lane < 12

    def group_body(g, _):
        off = tok_base + g * IDX_PER_GROUP
        pltpu.sync_copy(text_hbm.at[pl.ds(off, IDX_PER_GROUP)], idx_v)

        def fire(j, carry):
            pltpu.async_copy(
                p_hbm.at[idx_v.at[pl.ds(j * STREAM_LEN, STREAM_LEN)]],
                rows_v.at[pl.ds(j * STREAM_LEN, STREAM_LEN)],
                sem,
            )
            return carry

        lax.fori_loop(0, NSTREAMS, fire, 0)

        def drain(j, carry):
            pltpu.make_async_copy(
                p_hbm.at[idx_v.at[pl.ds(j * STREAM_LEN, STREAM_LEN)]],
                rows_v.at[pl.ds(j * STREAM_LEN, STREAM_LEN)],
                sem,
            ).wait()
            return carry

        lax.fori_loop(0, NSTREAMS, drain, 0)

        def quad_body(q, carry):
            accs = []
            for rr in range(4):
                row0 = (q * 4 + rr) * SEQ

                def chunk(kb, acc4, row0=row0):
                    a0, a1, a2, a3 = acc4
                    base = row0 + kb * 20
                    for u in range(0, 20, 4):
                        a0 = a0 + rows_v[base + u]
                        a1 = a1 + rows_v[base + u + 1]
                        a2 = a2 + rows_v[base + u + 2]
                        a3 = a3 + rows_v[base + u + 3]
                    return (a0, a1, a2, a3)

                zero = jnp.zeros((LANES,), jnp.float32)
                a0, a1, a2, a3 = lax.fori_loop(
                    0, SEQ // 20, chunk, (zero, zero, zero, zero)
                )
                accs.append((a0 + a1) + (a2 + a3))
            out16 = jnp.where(m0, accs[0],
                              jnp.where(m1, accs[1],
                                        jnp.where(m2, accs[2], accs[3])))
            out_v[pl.ds(g * (GROUP * NUM_CLASSES) + q * LANES, LANES)] = out16
            return carry

        lax.fori_loop(0, GROUP // 4, quad_body, 0)
        return _

    lax.fori_loop(0, NGROUPS, group_body, 0)
    pltpu.sync_copy(
        out_v,
        out_hbm.at[pl.ds(wid * (ROWS_PER_W * NUM_CLASSES),
                         ROWS_PER_W * NUM_CLASSES)],
    )


def kernel(x_batch, emb_table, fc_w, fc_b):
    text = x_batch[:, 1:].astype(jnp.int32).reshape(-1)
    scale = jnp.float32(1.0 / SEQ)
    w16 = jnp.tile(fc_w.T * scale, (1, 4))
    b16 = jnp.tile(fc_b * scale, 4).reshape(1, LANES)
    p16 = _project(emb_table, w16, b16)
    out = _pool_kernel(text, p16)
    return out.reshape(BATCH, NUM_CLASSES)
